# baseline (speedup 1.0000x reference)
import functools

import jax
import jax.numpy as jnp
from jax import lax
from jax.experimental import pallas as pl
from jax.experimental.pallas import tpu as pltpu
from jax.experimental.pallas import tpu_sc as plsc

_NC = 2
_NS = 16
_NW = _NC * _NS
_L = 16

_V = 1000000
_D = 32
_MAXLEN = 200
_BATCH = 4096

_VT_BLK = 6
_TOK_BLK = 128 * _VT_BLK
_ROW_BLK = _TOK_BLK // 4
_NBLK = 7812 // _VT_BLK
_A_ITER = 21

_mesh = plsc.VectorSubcoreMesh(core_axis_name="c", subcore_axis_name="s")


def _iota16():
  return jnp.arange(16, dtype=jnp.int32)


def _full16(v):
  return jnp.full((16,), v, dtype=jnp.int32)


@functools.partial(
    pl.kernel,
    out_type=jax.ShapeDtypeStruct((_V // 4, 128), jnp.float32),
    mesh=_mesh,
    scratch_types=[
        pltpu.VMEM((2, 32, _TOK_BLK), jnp.float32),
        pltpu.VMEM((2, _ROW_BLK, 128), jnp.float32),
        pltpu.VMEM((32, 64), jnp.float32),
        pltpu.VMEM((16, 128), jnp.float32),
        [pltpu.SemaphoreType.DMA] * 2,
        [pltpu.SemaphoreType.DMA] * 2,
    ],
    compiler_params=pltpu.CompilerParams(needs_layout_passes=False),
)
def _stage_a(tokt_hbm, out_hbm, in_v, out_v, tail_in, tail_out, isems, osems):
  wid = lax.axis_index("s") * _NC + lax.axis_index("c")

  def fire_in(j, b):
    col = pl.multiple_of(j * _TOK_BLK, 128)
    for ct in range(4):
      pltpu.async_copy(
          tokt_hbm.at[pl.ds(8 * ct, 8), pl.ds(col, _TOK_BLK)],
          in_v.at[b, pl.ds(8 * ct, 8)], isems[b])

  def wait_in(b):
    for _ct in range(4):
      pltpu.make_async_copy(
          tokt_hbm.at[pl.ds(0, 8), pl.ds(0, _TOK_BLK)],
          in_v.at[b, pl.ds(0, 8)], isems[b]).wait()

  def wait_out(b):
    pltpu.make_async_copy(out_v.at[b], out_hbm.at[pl.ds(0, _ROW_BLK)],
                          osems[b]).wait()

  def blk_of(j):
    return wid + _NW * j

  @pl.when(blk_of(0) < _NBLK)
  def _():
    fire_in(blk_of(0), 0)

  @pl.when(blk_of(1) < _NBLK)
  def _():
    fire_in(blk_of(1), 1)

  def body(g, carry):
    for b in range(2):
      j = g * 2 + b
      blk = blk_of(j)

      @pl.when(blk < _NBLK)
      def _():
        wait_in(b)

        @pl.when(j >= 2)
        def _():
          wait_out(b)

        def row(r, c2):
          for q in range(4):
            t = 4 * r + q
            tvec = _full16(t)
            for h in range(2):
              vals = plsc.load_gather(in_v.at[b],
                                      [_iota16() + (16 * h), tvec])
              out_v[b, r, pl.ds(q * 32 + 16 * h, 16)] = vals
          return c2

        lax.fori_loop(0, _ROW_BLK, row, 0)
        pltpu.async_copy(
            out_v.at[b],
            out_hbm.at[pl.ds(pl.multiple_of(blk * _ROW_BLK, 8), _ROW_BLK)],
            osems[b])

        @pl.when(blk_of(j + 2) < _NBLK)
        def _():
          fire_in(blk_of(j + 2), b)

    return carry

  lax.fori_loop(0, _A_ITER, body, 0)
  wait_out(0)
  wait_out(1)

  @pl.when(wid == _NW - 1)
  def _():
    pltpu.sync_copy(tokt_hbm.at[:, pl.ds(7812 * 128, 64)], tail_in)
    for r in range(16):
      for q in range(4):
        t = 4 * r + q
        tvec = _full16(t)
        for h in range(2):
          vals = plsc.load_gather(tail_in, [_iota16() + (16 * h), tvec])
          tail_out[r, pl.ds(q * 32 + 16 * h, 16)] = vals
    pltpu.sync_copy(tail_out, out_hbm.at[pl.ds(249984, 16)])


_UG = 4
_GROUPS = (_MAXLEN * (_BATCH // 128)) // _UG // _NW
_GROW = 128 * _UG


@functools.partial(
    pl.kernel,
    out_type=jax.ShapeDtypeStruct((_MAXLEN, 4, _BATCH // 128, 8, 128),
                                  jnp.float32),
    mesh=_mesh,
    scratch_types=[
        pltpu.VMEM((_GROUPS * _GROW,), jnp.int32),
        pltpu.VMEM((2, _GROW, _D), jnp.float32),
        pltpu.VMEM((2, _UG, 4, 8, 128), jnp.float32),
        pltpu.VMEM((_MAXLEN, _D), jnp.float32),
        [pltpu.SemaphoreType.DMA] * 2,
        [pltpu.SemaphoreType.DMA] * 2,
    ],
    compiler_params=pltpu.CompilerParams(use_tc_tiling_on_sc=False, needs_layout_passes=False),
)
def _stage_b(xf_hbm, tok_hbm, pos_hbm, out_hbm, idx_v, rows_v, tile_v, pos_v,
             gsems, osems):
  wid = lax.axis_index("s") * _NC + lax.axis_index("c")
  ubase = wid * _GROUPS * _UG
  pltpu.sync_copy(pos_hbm, pos_v)
  pltpu.sync_copy(xf_hbm.at[pl.ds(ubase * 128, _GROUPS * _GROW)], idx_v)

  def fire_gather(gg, b):
    pltpu.async_copy(tok_hbm.at[idx_v.at[pl.ds(gg * _GROW, _GROW)]],
                     rows_v.at[b], gsems[b])

  def wait_gather(b):
    pltpu.make_async_copy(tok_hbm.at[idx_v.at[pl.ds(0, _GROW)]],
                          rows_v.at[b], gsems[b]).wait()

  def wait_outs(b):
    for _n in range(_UG * 4):
      pltpu.make_async_copy(tile_v.at[b, 0, 0], out_hbm.at[0, 0, 0],
                            osems[b]).wait()

  fire_gather(0, 0)
  fire_gather(1, 1)

  def body(g, carry):
    for b in range(2):
      gg = g * 2 + b
      wait_gather(b)

      @pl.when(gg >= 2)
      def _():
        wait_outs(b)

      for j in range(_UG):
        u = ubase + gg * _UG + j
        l = u // (_BATCH // 128)
        bt = u % (_BATCH // 128)
        lvec = _full16(l)
        for c in range(_D):
          pv = plsc.load_gather(pos_v, [lvec, _full16(c)])
          cvec = _full16(c)
          for tb in range(8):
            tvec = _iota16() + (j * 128 + 16 * tb)
            vals = plsc.load_gather(rows_v.at[b], [tvec, cvec]) + pv
            tile_v[b, j, c // 8, c % 8, pl.ds(16 * tb, 16)] = vals
        for ct in range(4):
          pltpu.async_copy(tile_v.at[b, j, ct], out_hbm.at[l, ct, bt],
                           osems[b])

      @pl.when(gg + 2 < _GROUPS)
      def _():
        fire_gather(gg + 2, b)

    return carry

  lax.fori_loop(0, _GROUPS // 2, body, 0)
  for b in range(2):
    wait_outs(b)


def kernel(x, token_table, pos_table):
  tokt = token_table.T
  tok_lin4 = _stage_a(tokt)
  tok_lin = tok_lin4.reshape(_V, _D)
  xf = x.T.reshape(_BATCH * _MAXLEN).astype(jnp.int32)
  out5 = _stage_b(xf, tok_lin, pos_table)
  return out5.transpose(2, 4, 0, 1, 3).reshape(_BATCH, _MAXLEN, _D)


# speedup vs baseline: 1.4526x; 1.4526x over previous
import functools

import jax
import jax.numpy as jnp
from jax import lax
from jax.experimental import pallas as pl
from jax.experimental.pallas import tpu as pltpu
from jax.experimental.pallas import tpu_sc as plsc

_NC = 2
_NS = 16
_NW = _NC * _NS
_L = 16

_V = 1000000
_D = 32
_MAXLEN = 200
_BATCH = 4096

_VT_BLK = 6
_TOK_BLK = 128 * _VT_BLK
_ROW_BLK = _TOK_BLK // 4
_NBLK = 7812 // _VT_BLK
_A_ITER = 21

_mesh = plsc.VectorSubcoreMesh(core_axis_name="c", subcore_axis_name="s")


def _iota16():
  return jnp.arange(16, dtype=jnp.int32)


def _full16(v):
  return jnp.full((16,), v, dtype=jnp.int32)


@functools.partial(
    pl.kernel,
    out_type=jax.ShapeDtypeStruct((_V // 4, 128), jnp.float32),
    mesh=_mesh,
    scratch_types=[
        pltpu.VMEM((2, 32, _TOK_BLK), jnp.float32),
        pltpu.VMEM((2, _ROW_BLK, 128), jnp.float32),
        pltpu.VMEM((32, 64), jnp.float32),
        pltpu.VMEM((16, 128), jnp.float32),
        [pltpu.SemaphoreType.DMA] * 2,
        [pltpu.SemaphoreType.DMA] * 2,
    ],
    compiler_params=pltpu.CompilerParams(needs_layout_passes=False),
)
def _stage_a(tokt_hbm, out_hbm, in_v, out_v, tail_in, tail_out, isems, osems):
  wid = lax.axis_index("s") * _NC + lax.axis_index("c")

  def fire_in(j, b):
    col = pl.multiple_of(j * _TOK_BLK, 128)
    for ct in range(4):
      pltpu.async_copy(
          tokt_hbm.at[pl.ds(8 * ct, 8), pl.ds(col, _TOK_BLK)],
          in_v.at[b, pl.ds(8 * ct, 8)], isems[b])

  def wait_in(b):
    for _ct in range(4):
      pltpu.make_async_copy(
          tokt_hbm.at[pl.ds(0, 8), pl.ds(0, _TOK_BLK)],
          in_v.at[b, pl.ds(0, 8)], isems[b]).wait()

  def wait_out(b):
    pltpu.make_async_copy(out_v.at[b], out_hbm.at[pl.ds(0, _ROW_BLK)],
                          osems[b]).wait()

  def blk_of(j):
    return wid + _NW * j

  @pl.when(blk_of(0) < _NBLK)
  def _():
    fire_in(blk_of(0), 0)

  @pl.when(blk_of(1) < _NBLK)
  def _():
    fire_in(blk_of(1), 1)

  def body(g, carry):
    for b in range(2):
      j = g * 2 + b
      blk = blk_of(j)

      @pl.when(blk < _NBLK)
      def _():
        wait_in(b)

        @pl.when(j >= 2)
        def _():
          wait_out(b)

        def loads(r):
          vs = []
          for q in range(4):
            tvec = _full16(4 * r + q)
            for h in range(2):
              vs.append(
                  plsc.load_gather(in_v.at[b], [_iota16() + (16 * h), tvec]))
          return vs

        def stores(r, vs):
          for q in range(4):
            for h in range(2):
              out_v[b, r, pl.ds(q * 32 + 16 * h, 16)] = vs[2 * q + h]

        def rowstep(i, c2):
          r0 = i * 2
          vs0 = loads(r0)
          vs1 = loads(r0 + 1)
          stores(r0, vs0)
          stores(r0 + 1, vs1)
          return c2

        lax.fori_loop(0, _ROW_BLK // 2, rowstep, 0)
        pltpu.async_copy(
            out_v.at[b],
            out_hbm.at[pl.ds(pl.multiple_of(blk * _ROW_BLK, 8), _ROW_BLK)],
            osems[b])

        @pl.when(blk_of(j + 2) < _NBLK)
        def _():
          fire_in(blk_of(j + 2), b)

    return carry

  lax.fori_loop(0, _A_ITER, body, 0)
  wait_out(0)
  wait_out(1)

  @pl.when(wid == _NW - 1)
  def _():
    pltpu.sync_copy(tokt_hbm.at[:, pl.ds(7812 * 128, 64)], tail_in)
    for r in range(16):
      for q in range(4):
        t = 4 * r + q
        tvec = _full16(t)
        for h in range(2):
          vals = plsc.load_gather(tail_in, [_iota16() + (16 * h), tvec])
          tail_out[r, pl.ds(q * 32 + 16 * h, 16)] = vals
    pltpu.sync_copy(tail_out, out_hbm.at[pl.ds(249984, 16)])


_UG = 4
_GROUPS = (_MAXLEN * (_BATCH // 128)) // _UG // _NW
_GROW = 128 * _UG


@functools.partial(
    pl.kernel,
    out_type=jax.ShapeDtypeStruct((_MAXLEN, 4, _BATCH // 128, 8, 128),
                                  jnp.float32),
    mesh=_mesh,
    scratch_types=[
        pltpu.VMEM((_GROUPS * _GROW,), jnp.int32),
        pltpu.VMEM((2, _GROW, _D), jnp.float32),
        pltpu.VMEM((2, _UG, 4, 8, 128), jnp.float32),
        pltpu.VMEM((_MAXLEN, _D), jnp.float32),
        [pltpu.SemaphoreType.DMA] * 2,
        [pltpu.SemaphoreType.DMA] * 2,
    ],
    compiler_params=pltpu.CompilerParams(use_tc_tiling_on_sc=False, needs_layout_passes=False),
)
def _stage_b(xf_hbm, tok_hbm, pos_hbm, out_hbm, idx_v, rows_v, tile_v, pos_v,
             gsems, osems):
  wid = lax.axis_index("s") * _NC + lax.axis_index("c")
  ubase = wid * _GROUPS * _UG
  pltpu.sync_copy(pos_hbm, pos_v)
  pltpu.sync_copy(xf_hbm.at[pl.ds(ubase * 128, _GROUPS * _GROW)], idx_v)

  def fire_gather(gg, b):
    pltpu.async_copy(tok_hbm.at[idx_v.at[pl.ds(gg * _GROW, _GROW)]],
                     rows_v.at[b], gsems[b])

  def wait_gather(b):
    pltpu.make_async_copy(tok_hbm.at[idx_v.at[pl.ds(0, _GROW)]],
                          rows_v.at[b], gsems[b]).wait()

  def wait_outs(b):
    for _n in range(_UG * 4):
      pltpu.make_async_copy(tile_v.at[b, 0, 0], out_hbm.at[0, 0, 0],
                            osems[b]).wait()

  fire_gather(0, 0)
  fire_gather(1, 1)

  def body(g, carry):
    for b in range(2):
      gg = g * 2 + b
      wait_gather(b)

      @pl.when(gg >= 2)
      def _():
        wait_outs(b)

      for j in range(_UG):
        u = ubase + gg * _UG + j
        l = u // (_BATCH // 128)
        bt = u % (_BATCH // 128)
        lvec = _full16(l)

        def loads(c):
          pv = plsc.load_gather(pos_v, [lvec, _full16(c)])
          cvec = _full16(c)
          vs = []
          for tb in range(8):
            tvec = _iota16() + (j * 128 + 16 * tb)
            vs.append(plsc.load_gather(rows_v.at[b], [tvec, cvec]))
          return pv, vs

        def stores(c, pv, vs):
          for tb in range(8):
            tile_v[b, j, c // 8, c % 8, pl.ds(16 * tb, 16)] = vs[tb] + pv

        prev = loads(0)
        for c in range(1, _D):
          cur = loads(c)
          stores(c - 1, *prev)
          prev = cur
        stores(_D - 1, *prev)
        for ct in range(4):
          pltpu.async_copy(tile_v.at[b, j, ct], out_hbm.at[l, ct, bt],
                           osems[b])

      @pl.when(gg + 2 < _GROUPS)
      def _():
        fire_gather(gg + 2, b)

    return carry

  lax.fori_loop(0, _GROUPS // 2, body, 0)
  for b in range(2):
    wait_outs(b)


def kernel(x, token_table, pos_table):
  tokt = token_table.T
  tok_lin4 = _stage_a(tokt)
  tok_lin = tok_lin4.reshape(_V, _D)
  xf = x.T.reshape(_BATCH * _MAXLEN).astype(jnp.int32)
  out5 = _stage_b(xf, tok_lin, pos_table)
  return out5.transpose(2, 4, 0, 1, 3).reshape(_BATCH, _MAXLEN, _D)


# speedup vs baseline: 1.4557x; 1.0022x over previous
import functools

import jax
import jax.numpy as jnp
from jax import lax
from jax.experimental import pallas as pl
from jax.experimental.pallas import tpu as pltpu
from jax.experimental.pallas import tpu_sc as plsc

_NC = 2
_NS = 16
_NW = _NC * _NS
_L = 16

_V = 1000000
_D = 32
_MAXLEN = 200
_BATCH = 4096

_VT_BLK = 6
_TOK_BLK = 128 * _VT_BLK
_ROW_BLK = _TOK_BLK // 4
_NBLK = 7812 // _VT_BLK
_A_ITER = 21

_mesh = plsc.VectorSubcoreMesh(core_axis_name="c", subcore_axis_name="s")


def _iota16():
  return jnp.arange(16, dtype=jnp.int32)


def _full16(v):
  return jnp.full((16,), v, dtype=jnp.int32)


@functools.partial(
    pl.kernel,
    out_type=jax.ShapeDtypeStruct((_V // 4, 128), jnp.float32),
    mesh=_mesh,
    scratch_types=[
        pltpu.VMEM((2, 32, _TOK_BLK), jnp.float32),
        pltpu.VMEM((2, _ROW_BLK, 128), jnp.float32),
        pltpu.VMEM((32, 64), jnp.float32),
        pltpu.VMEM((16, 128), jnp.float32),
        [pltpu.SemaphoreType.DMA] * 2,
        [pltpu.SemaphoreType.DMA] * 2,
    ],
    compiler_params=pltpu.CompilerParams(needs_layout_passes=False),
)
def _stage_a(tokt_hbm, out_hbm, in_v, out_v, tail_in, tail_out, isems, osems):
  wid = lax.axis_index("s") * _NC + lax.axis_index("c")

  def fire_in(j, b):
    for ct in range(4):
      for vt in range(_VT_BLK):
        col = pl.multiple_of(j * _TOK_BLK + vt * 128, 128)
        pltpu.async_copy(
            tokt_hbm.at[pl.ds(8 * ct, 8), pl.ds(col, 128)],
            in_v.at[b, pl.ds(8 * ct, 8), pl.ds(vt * 128, 128)], isems[b])

  def wait_in(b):
    for _n in range(4 * _VT_BLK):
      pltpu.make_async_copy(
          tokt_hbm.at[pl.ds(0, 8), pl.ds(0, 128)],
          in_v.at[b, pl.ds(0, 8), pl.ds(0, 128)], isems[b]).wait()

  def wait_out(b):
    pltpu.make_async_copy(out_v.at[b], out_hbm.at[pl.ds(0, _ROW_BLK)],
                          osems[b]).wait()

  def blk_of(j):
    return wid + _NW * j

  @pl.when(blk_of(0) < _NBLK)
  def _():
    fire_in(blk_of(0), 0)

  @pl.when(blk_of(1) < _NBLK)
  def _():
    fire_in(blk_of(1), 1)

  def body(g, carry):
    for b in range(2):
      j = g * 2 + b
      blk = blk_of(j)

      @pl.when(blk < _NBLK)
      def _():
        wait_in(b)

        @pl.when(j >= 2)
        def _():
          wait_out(b)

        def loads(r):
          vs = []
          for q in range(4):
            tvec = _full16(4 * r + q)
            for h in range(2):
              vs.append(
                  plsc.load_gather(in_v.at[b], [_iota16() + (16 * h), tvec]))
          return vs

        def stores(r, vs):
          for q in range(4):
            for h in range(2):
              out_v[b, r, pl.ds(q * 32 + 16 * h, 16)] = vs[2 * q + h]

        def rowstep(i, c2):
          r0 = i * 2
          vs0 = loads(r0)
          vs1 = loads(r0 + 1)
          stores(r0, vs0)
          stores(r0 + 1, vs1)
          return c2

        lax.fori_loop(0, _ROW_BLK // 2, rowstep, 0)
        pltpu.async_copy(
            out_v.at[b],
            out_hbm.at[pl.ds(pl.multiple_of(blk * _ROW_BLK, 8), _ROW_BLK)],
            osems[b])

        @pl.when(blk_of(j + 2) < _NBLK)
        def _():
          fire_in(blk_of(j + 2), b)

    return carry

  lax.fori_loop(0, _A_ITER, body, 0)
  wait_out(0)
  wait_out(1)

  @pl.when(wid == _NW - 1)
  def _():
    pltpu.sync_copy(tokt_hbm.at[:, pl.ds(7812 * 128, 64)], tail_in)
    for r in range(16):
      for q in range(4):
        t = 4 * r + q
        tvec = _full16(t)
        for h in range(2):
          vals = plsc.load_gather(tail_in, [_iota16() + (16 * h), tvec])
          tail_out[r, pl.ds(q * 32 + 16 * h, 16)] = vals
    pltpu.sync_copy(tail_out, out_hbm.at[pl.ds(249984, 16)])


_UG = 4
_GROUPS = (_MAXLEN * (_BATCH // 128)) // _UG // _NW
_GROW = 128 * _UG


@functools.partial(
    pl.kernel,
    out_type=jax.ShapeDtypeStruct((_MAXLEN, 4, _BATCH // 128, 8, 128),
                                  jnp.float32),
    mesh=_mesh,
    scratch_types=[
        pltpu.VMEM((_GROUPS * _GROW,), jnp.int32),
        pltpu.VMEM((2, _GROW, _D), jnp.float32),
        pltpu.VMEM((2, _UG, 4, 8, 128), jnp.float32),
        pltpu.VMEM((_MAXLEN, _D), jnp.float32),
        [pltpu.SemaphoreType.DMA] * 2,
        [pltpu.SemaphoreType.DMA] * 2,
    ],
    compiler_params=pltpu.CompilerParams(use_tc_tiling_on_sc=False, needs_layout_passes=False),
)
def _stage_b(xf_hbm, tok_hbm, pos_hbm, out_hbm, idx_v, rows_v, tile_v, pos_v,
             gsems, osems):
  wid = lax.axis_index("s") * _NC + lax.axis_index("c")
  ubase = wid * _GROUPS * _UG
  pltpu.sync_copy(pos_hbm, pos_v)
  pltpu.sync_copy(xf_hbm.at[pl.ds(ubase * 128, _GROUPS * _GROW)], idx_v)

  def fire_gather(gg, b):
    pltpu.async_copy(tok_hbm.at[idx_v.at[pl.ds(gg * _GROW, _GROW)]],
                     rows_v.at[b], gsems[b])

  def wait_gather(b):
    pltpu.make_async_copy(tok_hbm.at[idx_v.at[pl.ds(0, _GROW)]],
                          rows_v.at[b], gsems[b]).wait()

  def wait_outs(b):
    for _n in range(_UG):
      pltpu.make_async_copy(tile_v.at[b, 0], out_hbm.at[0, :, 0],
                            osems[b]).wait()

  fire_gather(0, 0)
  fire_gather(1, 1)

  def body(g, carry):
    for b in range(2):
      gg = g * 2 + b
      wait_gather(b)

      @pl.when(gg >= 2)
      def _():
        wait_outs(b)

      for j in range(_UG):
        u = ubase + gg * _UG + j
        l = u // (_BATCH // 128)
        bt = u % (_BATCH // 128)
        lvec = _full16(l)

        def loads(c):
          pv = plsc.load_gather(pos_v, [lvec, _full16(c)])
          cvec = _full16(c)
          vs = []
          for tb in range(8):
            tvec = _iota16() + (j * 128 + 16 * tb)
            vs.append(plsc.load_gather(rows_v.at[b], [tvec, cvec]))
          return pv, vs

        def stores(c, pv, vs):
          for tb in range(8):
            tile_v[b, j, c // 8, c % 8, pl.ds(16 * tb, 16)] = vs[tb] + pv

        prev = loads(0)
        for c in range(1, _D):
          cur = loads(c)
          stores(c - 1, *prev)
          prev = cur
        stores(_D - 1, *prev)
        pltpu.async_copy(tile_v.at[b, j], out_hbm.at[l, :, bt], osems[b])

      @pl.when(gg + 2 < _GROUPS)
      def _():
        fire_gather(gg + 2, b)

    return carry

  lax.fori_loop(0, _GROUPS // 2, body, 0)
  for b in range(2):
    wait_outs(b)


def kernel(x, token_table, pos_table):
  tokt = token_table.T
  tok_lin4 = _stage_a(tokt)
  tok_lin = tok_lin4.reshape(_V, _D)
  xf = x.T.reshape(_BATCH * _MAXLEN).astype(jnp.int32)
  out5 = _stage_b(xf, tok_lin, pos_table)
  return out5.transpose(2, 4, 0, 1, 3).reshape(_BATCH, _MAXLEN, _D)


# speedup vs baseline: 1.9462x; 1.3369x over previous
import functools

import jax
import jax.numpy as jnp
from jax import lax
from jax.experimental import pallas as pl
from jax.experimental.pallas import tpu as pltpu
from jax.experimental.pallas import tpu_sc as plsc

_NC = 2
_NS = 16
_NW = _NC * _NS
_L = 16

_V = 1000000
_D = 32
_MAXLEN = 200
_BATCH = 4096

_VT_BLK = 6
_TOK_BLK = 128 * _VT_BLK
_ROW_BLK = _TOK_BLK // 4
_NBLK = 7812 // _VT_BLK
_A_ITER = 21

_mesh = plsc.VectorSubcoreMesh(core_axis_name="c", subcore_axis_name="s")


def _iota16():
  return jnp.arange(16, dtype=jnp.int32)


def _full16(v):
  return jnp.full((16,), v, dtype=jnp.int32)


@functools.partial(
    pl.kernel,
    out_type=jax.ShapeDtypeStruct((_V // 4, 128), jnp.float32),
    mesh=_mesh,
    scratch_types=[
        pltpu.VMEM((2, 32, _TOK_BLK), jnp.float32),
        pltpu.VMEM((2, _ROW_BLK, 128), jnp.float32),
        pltpu.VMEM((32, 64), jnp.float32),
        pltpu.VMEM((16, 128), jnp.float32),
        [pltpu.SemaphoreType.DMA] * 2,
        [pltpu.SemaphoreType.DMA] * 2,
    ],
    compiler_params=pltpu.CompilerParams(needs_layout_passes=False),
)
def _stage_a(tokt_hbm, out_hbm, in_v, out_v, tail_in, tail_out, isems, osems):
  wid = lax.axis_index("s") * _NC + lax.axis_index("c")

  def fire_in(j, b):
    for ct in range(4):
      for vt in range(_VT_BLK):
        col = pl.multiple_of(j * _TOK_BLK + vt * 128, 128)
        pltpu.async_copy(
            tokt_hbm.at[pl.ds(8 * ct, 8), pl.ds(col, 128)],
            in_v.at[b, pl.ds(8 * ct, 8), pl.ds(vt * 128, 128)], isems[b])

  def wait_in(b):
    for _n in range(4 * _VT_BLK):
      pltpu.make_async_copy(
          tokt_hbm.at[pl.ds(0, 8), pl.ds(0, 128)],
          in_v.at[b, pl.ds(0, 8), pl.ds(0, 128)], isems[b]).wait()

  def wait_out(b):
    pltpu.make_async_copy(out_v.at[b], out_hbm.at[pl.ds(0, _ROW_BLK)],
                          osems[b]).wait()

  def blk_of(j):
    return wid + _NW * j

  @pl.when(blk_of(0) < _NBLK)
  def _():
    fire_in(blk_of(0), 0)

  @pl.when(blk_of(1) < _NBLK)
  def _():
    fire_in(blk_of(1), 1)

  def body(g, carry):
    for b in range(2):
      j = g * 2 + b
      blk = blk_of(j)

      @pl.when(blk < _NBLK)
      def _():
        wait_in(b)

        @pl.when(j >= 2)
        def _():
          wait_out(b)

        def loads(r):
          vs = []
          for q in range(4):
            tvec = _full16(4 * r + q)
            for h in range(2):
              vs.append(
                  plsc.load_gather(in_v.at[b], [_iota16() + (16 * h), tvec]))
          return vs

        def stores(r, vs):
          for q in range(4):
            for h in range(2):
              out_v[b, r, pl.ds(q * 32 + 16 * h, 16)] = vs[2 * q + h]

        def rowstep(i, c2):
          r0 = i * 2
          vs0 = loads(r0)
          vs1 = loads(r0 + 1)
          stores(r0, vs0)
          stores(r0 + 1, vs1)
          return c2

        lax.fori_loop(0, _ROW_BLK // 2, rowstep, 0)
        pltpu.async_copy(
            out_v.at[b],
            out_hbm.at[pl.ds(pl.multiple_of(blk * _ROW_BLK, 8), _ROW_BLK)],
            osems[b])

        @pl.when(blk_of(j + 2) < _NBLK)
        def _():
          fire_in(blk_of(j + 2), b)

    return carry

  lax.fori_loop(0, _A_ITER, body, 0)
  wait_out(0)
  wait_out(1)

  @pl.when(wid == _NW - 1)
  def _():
    pltpu.sync_copy(tokt_hbm.at[:, pl.ds(7812 * 128, 64)], tail_in)
    for r in range(16):
      for q in range(4):
        t = 4 * r + q
        tvec = _full16(t)
        for h in range(2):
          vals = plsc.load_gather(tail_in, [_iota16() + (16 * h), tvec])
          tail_out[r, pl.ds(q * 32 + 16 * h, 16)] = vals
    pltpu.sync_copy(tail_out, out_hbm.at[pl.ds(249984, 16)])


_UG = 2
_GROUPS = (_MAXLEN * (_BATCH // 128)) // _UG // _NW
_GROW = 128 * _UG


@functools.partial(
    pl.kernel,
    out_type=jax.ShapeDtypeStruct((_MAXLEN, 4, _BATCH // 128, 8, 128),
                                  jnp.float32),
    mesh=_mesh,
    scratch_types=[
        pltpu.VMEM((_GROUPS * _GROW,), jnp.int32),
        pltpu.VMEM((2, _GROW, _D), jnp.float32),
        pltpu.VMEM((2, _UG, 32, 129), jnp.float32),
        pltpu.VMEM((_MAXLEN, _D), jnp.float32),
        [pltpu.SemaphoreType.DMA] * 2,
        [pltpu.SemaphoreType.DMA] * 2,
    ],
    compiler_params=pltpu.CompilerParams(use_tc_tiling_on_sc=False, needs_layout_passes=False),
)
def _stage_b(xf_hbm, tok_hbm, pos_hbm, out_hbm, idx_v, rows_v, tile_v, pos_v,
             gsems, osems):
  wid = lax.axis_index("s") * _NC + lax.axis_index("c")
  ubase = wid * _GROUPS * _UG
  pltpu.sync_copy(pos_hbm, pos_v)
  pltpu.sync_copy(xf_hbm.at[pl.ds(ubase * 128, _GROUPS * _GROW)], idx_v)

  def fire_gather(gg, b):
    pltpu.async_copy(tok_hbm.at[idx_v.at[pl.ds(gg * _GROW, _GROW)]],
                     rows_v.at[b], gsems[b])

  def wait_gather(b):
    pltpu.make_async_copy(tok_hbm.at[idx_v.at[pl.ds(0, _GROW)]],
                          rows_v.at[b], gsems[b]).wait()

  def wait_outs(b):
    for _n in range(_UG * 4):
      pltpu.make_async_copy(tile_v.at[b, 0, pl.ds(0, 8), pl.ds(0, 128)],
                            out_hbm.at[0, 0, 0], osems[b]).wait()

  fire_gather(0, 0)
  fire_gather(1, 1)

  def body(g, carry):
    for b in range(2):
      gg = g * 2 + b
      wait_gather(b)

      @pl.when(gg >= 2)
      def _():
        wait_outs(b)

      for j in range(_UG):
        u = ubase + gg * _UG + j
        l = u // (_BATCH // 128)
        bt = u % (_BATCH // 128)
        pv = [pos_v[l, pl.ds(0, 16)], pos_v[l, pl.ds(16, 16)]]

        def loads(t):
          r = j * 128 + t
          return [rows_v[b, r, pl.ds(0, 16)] + pv[0],
                  rows_v[b, r, pl.ds(16, 16)] + pv[1]]

        def stores(t, vs):
          for h in range(2):
            plsc.store_scatter(tile_v.at[b, j],
                               [_iota16() + (16 * h), _full16(t)], vs[h])

        prev = loads(0)
        for t in range(1, 128):
          cur = loads(t)
          stores(t - 1, prev)
          prev = cur
        stores(127, prev)
        for ct in range(4):
          pltpu.async_copy(tile_v.at[b, j, pl.ds(8 * ct, 8), pl.ds(0, 128)],
                           out_hbm.at[l, ct, bt], osems[b])

      @pl.when(gg + 2 < _GROUPS)
      def _():
        fire_gather(gg + 2, b)

    return carry

  lax.fori_loop(0, _GROUPS // 2, body, 0)
  for b in range(2):
    wait_outs(b)


def kernel(x, token_table, pos_table):
  tokt = token_table.T
  tok_lin4 = _stage_a(tokt)
  tok_lin = tok_lin4.reshape(_V, _D)
  xf = x.T.reshape(_BATCH * _MAXLEN).astype(jnp.int32)
  out5 = _stage_b(xf, tok_lin, pos_table)
  return out5.transpose(2, 4, 0, 1, 3).reshape(_BATCH, _MAXLEN, _D)


# speedup vs baseline: 2.7041x; 1.3895x over previous
import functools

import jax
import jax.numpy as jnp
from jax import lax
from jax.experimental import pallas as pl
from jax.experimental.pallas import tpu as pltpu
from jax.experimental.pallas import tpu_sc as plsc

_NC = 2
_NS = 16
_NW = _NC * _NS
_L = 16

_V = 1000000
_D = 32
_MAXLEN = 200
_BATCH = 4096

_TBLK = 8192

_mesh = plsc.VectorSubcoreMesh(core_axis_name="c", subcore_axis_name="s")


def _iota16():
  return jnp.arange(16, dtype=jnp.int32)


def _full16(v):
  return jnp.full((16,), v, dtype=jnp.int32)


_NBLK_A = (_V + _TBLK - 1) // _TBLK
_VPAD = _NBLK_A * _TBLK


def _ta_body(tokt_ref, out_ref):
  blk = tokt_ref[...]
  blkt = blk.T
  for q in range(4):
    out_ref[:, pl.ds(32 * q, 32)] = blkt[2048 * q:2048 * (q + 1), :]


_stage_a = pl.pallas_call(
    _ta_body,
    grid=(_NBLK_A,),
    in_specs=[pl.BlockSpec((32, _TBLK), lambda i: (0, i))],
    out_specs=pl.BlockSpec((_TBLK // 4, 128), lambda i: (i, 0)),
    out_shape=jax.ShapeDtypeStruct((_VPAD // 4, 128), jnp.float32),
)

_UG = 4
_GROUPS = (_MAXLEN * (_BATCH // 128)) // _UG // _NW
_GROW = 128 * _UG


@functools.partial(
    pl.kernel,
    out_type=jax.ShapeDtypeStruct((_MAXLEN, 4, _BATCH // 128, 8, 128),
                                  jnp.float32),
    mesh=_mesh,
    scratch_types=[
        pltpu.VMEM((_GROUPS * _GROW,), jnp.int32),
        pltpu.VMEM((2, _GROW, _D), jnp.float32),
        pltpu.VMEM((2, _UG, 32, 129), jnp.float32),
        pltpu.VMEM((_MAXLEN, _D), jnp.float32),
        [pltpu.SemaphoreType.DMA] * 2,
        [pltpu.SemaphoreType.DMA] * 2,
    ],
    compiler_params=pltpu.CompilerParams(use_tc_tiling_on_sc=False, needs_layout_passes=False),
)
def _stage_b(xf_hbm, tok_hbm, pos_hbm, out_hbm, idx_v, rows_v, tile_v, pos_v,
             gsems, osems):
  wid = lax.axis_index("s") * _NC + lax.axis_index("c")
  ubase = wid * _GROUPS * _UG
  pltpu.sync_copy(pos_hbm, pos_v)
  pltpu.sync_copy(xf_hbm.at[pl.ds(ubase * 128, _GROUPS * _GROW)], idx_v)

  def remap(i, c2):
    for k in range(2):
      sl = pl.ds(i * 32 + k * 16, 16)
      v = idx_v[sl]
      idx_v[sl] = ((v & (-8192)) + ((v & 2047) << 2)) + ((v >> 11) & 3)
    return c2

  lax.fori_loop(0, _GROUPS * _GROW // 32, remap, 0)

  def fire_gather(gg, b):
    pltpu.async_copy(tok_hbm.at[idx_v.at[pl.ds(gg * _GROW, _GROW)]],
                     rows_v.at[b], gsems[b])

  def wait_gather(b):
    pltpu.make_async_copy(tok_hbm.at[idx_v.at[pl.ds(0, _GROW)]],
                          rows_v.at[b], gsems[b]).wait()

  def wait_outs(b):
    for _n in range(_UG * 4):
      pltpu.make_async_copy(tile_v.at[b, 0, pl.ds(0, 8), pl.ds(0, 128)],
                            out_hbm.at[0, 0, 0], osems[b]).wait()

  fire_gather(0, 0)
  fire_gather(1, 1)

  def body(g, carry):
    for b in range(2):
      gg = g * 2 + b
      wait_gather(b)

      @pl.when(gg >= 2)
      def _():
        wait_outs(b)

      for j in range(_UG):
        u = ubase + gg * _UG + j
        l = u // (_BATCH // 128)
        bt = u % (_BATCH // 128)
        pv = [pos_v[l, pl.ds(0, 16)], pos_v[l, pl.ds(16, 16)]]

        def loads(t):
          r = j * 128 + t
          return [rows_v[b, r, pl.ds(0, 16)] + pv[0],
                  rows_v[b, r, pl.ds(16, 16)] + pv[1]]

        def stores(t, vs):
          for h in range(2):
            plsc.store_scatter(tile_v.at[b, j],
                               [_iota16() + (16 * h), _full16(t)], vs[h])

        prev = loads(0)
        for t in range(1, 128):
          cur = loads(t)
          stores(t - 1, prev)
          prev = cur
        stores(127, prev)
        for ct in range(4):
          pltpu.async_copy(tile_v.at[b, j, pl.ds(8 * ct, 8), pl.ds(0, 128)],
                           out_hbm.at[l, ct, bt], osems[b])

      @pl.when(gg + 2 < _GROUPS)
      def _():
        fire_gather(gg + 2, b)

    return carry

  lax.fori_loop(0, _GROUPS // 2, body, 0)
  for b in range(2):
    wait_outs(b)


def kernel(x, token_table, pos_table):
  tokt = token_table.T
  tok_lin4 = _stage_a(tokt)
  tok_lin = tok_lin4.reshape(_VPAD, _D)
  xf = x.T.reshape(_BATCH * _MAXLEN).astype(jnp.int32)
  out5 = _stage_b(xf, tok_lin, pos_table)
  return out5.transpose(2, 4, 0, 1, 3).reshape(_BATCH, _MAXLEN, _D)
